# baseline (speedup 1.0000x reference)
import functools

import jax
import jax.numpy as jnp
from jax import lax
from jax.experimental import pallas as pl
from jax.experimental.pallas import tpu as pltpu
from jax.experimental.pallas import tpu_sc as plsc


@functools.lru_cache(maxsize=None)
def _make_sc_gather(V, H, B):
    info = plsc.get_sparse_core_info()
    NC, NS = info.num_cores, info.num_subcores
    NW = NC * NS
    assert H % info.num_lanes == 0 and B % (8 * NW) == 0
    b_per_w = B // NW
    mesh = plsc.VectorSubcoreMesh(core_axis_name="c", subcore_axis_name="s")

    @functools.partial(
        pl.kernel,
        mesh=mesh,
        out_type=jax.ShapeDtypeStruct((B, H), jnp.float32),
        scratch_types=[
            pltpu.VMEM((b_per_w,), jnp.int32),
            pltpu.VMEM((b_per_w, H), jnp.float32),
            pltpu.SemaphoreType.DMA,
        ],
        compiler_params=pltpu.CompilerParams(use_tc_tiling_on_sc=False),
    )
    def gather_kernel(idx_hbm, table_hbm, out_hbm, idx_v, rows_v, sem):
        wid = lax.axis_index("s") * NC + lax.axis_index("c")
        base = wid * b_per_w
        pltpu.sync_copy(idx_hbm.at[pl.ds(base, b_per_w)], idx_v)
        pltpu.async_copy(table_hbm.at[idx_v], rows_v, sem).wait()
        pltpu.sync_copy(rows_v, out_hbm.at[pl.ds(base, b_per_w)])

    return gather_kernel


def _proj_body(h_ref, w_ref, b_ref, out_ref):
    acc = lax.dot_general(
        h_ref[...], w_ref[...],
        (((1,), (1,)), ((), ())),
        preferred_element_type=jnp.float32,
    )
    out_ref[...] = acc + b_ref[...]


@functools.lru_cache(maxsize=None)
def _make_tc_proj(B, H, V, v_tile):
    grid = (pl.cdiv(V, v_tile),)
    return pl.pallas_call(
        _proj_body,
        grid=grid,
        in_specs=[
            pl.BlockSpec((B, H), lambda j: (0, 0)),
            pl.BlockSpec((v_tile, H), lambda j: (j, 0)),
            pl.BlockSpec((1, v_tile), lambda j: (0, j)),
        ],
        out_specs=pl.BlockSpec((B, v_tile), lambda j: (0, j)),
        out_shape=jax.ShapeDtypeStruct((B, V), jnp.float32),
        compiler_params=pltpu.CompilerParams(
            dimension_semantics=("arbitrary",),
        ),
    )


def kernel(input_ids, embed_table, proj_weight, proj_bias):
    B, = input_ids.shape
    V, H = embed_table.shape
    hidden = _make_sc_gather(V, H, B)(input_ids.astype(jnp.int32), embed_table)
    logits = _make_tc_proj(B, H, V, 1024)(
        hidden, proj_weight, proj_bias.reshape(1, V)
    )
    return logits


# speedup vs baseline: 1.0044x; 1.0044x over previous
import functools

import jax
import jax.numpy as jnp
from jax import lax
from jax.experimental import pallas as pl
from jax.experimental.pallas import tpu as pltpu
from jax.experimental.pallas import tpu_sc as plsc


@functools.lru_cache(maxsize=None)
def _make_sc_gather(V, H, B):
    info = plsc.get_sparse_core_info()
    NC, NS, L = info.num_cores, info.num_subcores, info.num_lanes
    NW = NC * NS
    assert H == 32 and B % (8 * NW) == 0 and V % 4 == 0
    b_per_w = B // NW
    n_grp = b_per_w // L
    mesh = plsc.VectorSubcoreMesh(core_axis_name="c", subcore_axis_name="s")

    @functools.partial(
        pl.kernel,
        mesh=mesh,
        out_type=jax.ShapeDtypeStruct((B, H), jnp.float32),
        scratch_types=[
            pltpu.VMEM((b_per_w,), jnp.int32),
            pltpu.VMEM((b_per_w,), jnp.int32),
            pltpu.VMEM((b_per_w, 4 * H), jnp.float32),
            pltpu.VMEM((b_per_w, H), jnp.float32),
            pltpu.SemaphoreType.DMA,
        ],
        compiler_params=pltpu.CompilerParams(needs_layout_passes=False),
    )
    def gather_kernel(idx_hbm, table_hbm, out_hbm, idx_v, grp_v, rows4_v,
                      rows_v, sem):
        wid = lax.axis_index("s") * NC + lax.axis_index("c")
        base = wid * b_per_w
        pltpu.sync_copy(idx_hbm.at[pl.ds(base, b_per_w)], idx_v)
        for g in range(n_grp):
            ids = idx_v[pl.ds(g * L, L)]
            grp_v[pl.ds(g * L, L)] = jax.lax.shift_right_logical(ids, 2)
        pltpu.async_copy(table_hbm.at[grp_v], rows4_v, sem).wait()
        for g in range(n_grp):
            ids = idx_v[pl.ds(g * L, L)]
            sub = jax.lax.bitwise_and(ids, 2**2 - 1)
            lane0 = sub * H
            row = jax.lax.iota(jnp.int32, L) + g * L
            for j in range(H):
                col = jnp.full((L,), j, jnp.int32)
                vals = plsc.load_gather(rows4_v, [row, lane0 + col])
                plsc.store_scatter(rows_v, [row, col], vals)
        pltpu.sync_copy(rows_v, out_hbm.at[pl.ds(base, b_per_w)])

    return gather_kernel


def _proj_body(h_ref, w_ref, b_ref, out_ref):
    acc = lax.dot_general(
        h_ref[...], w_ref[...],
        (((1,), (1,)), ((), ())),
        preferred_element_type=jnp.float32,
    )
    out_ref[...] = acc + b_ref[...]


@functools.lru_cache(maxsize=None)
def _make_tc_proj(B, H, V, v_tile):
    grid = (pl.cdiv(V, v_tile),)
    return pl.pallas_call(
        _proj_body,
        grid=grid,
        in_specs=[
            pl.BlockSpec((B, H), lambda j: (0, 0)),
            pl.BlockSpec((v_tile, H), lambda j: (j, 0)),
            pl.BlockSpec((1, v_tile), lambda j: (0, j)),
        ],
        out_specs=pl.BlockSpec((B, v_tile), lambda j: (0, j)),
        out_shape=jax.ShapeDtypeStruct((B, V), jnp.float32),
        compiler_params=pltpu.CompilerParams(
            dimension_semantics=("arbitrary",),
        ),
    )


def kernel(input_ids, embed_table, proj_weight, proj_bias):
    B, = input_ids.shape
    V, H = embed_table.shape
    table4 = embed_table.reshape(V // 4, 4 * H)
    hidden = _make_sc_gather(V, H, B)(input_ids.astype(jnp.int32), table4)
    logits = _make_tc_proj(B, H, V, 1024)(
        hidden, proj_weight, proj_bias.reshape(1, V)
    )
    return logits


# speedup vs baseline: 3.5284x; 3.5128x over previous
import functools

import jax
import jax.numpy as jnp
from jax import lax
from jax.experimental import pallas as pl
from jax.experimental.pallas import tpu as pltpu
from jax.experimental.pallas import tpu_sc as plsc


@functools.lru_cache(maxsize=None)
def _make_sc_gather(V, H, B):
    info = plsc.get_sparse_core_info()
    NC, NS, L = info.num_cores, info.num_subcores, info.num_lanes
    NW = NC * NS
    assert H == NW and B % L == 0
    n_grp = B // L
    mesh = plsc.VectorSubcoreMesh(core_axis_name="c", subcore_axis_name="s")

    @functools.partial(
        pl.kernel,
        mesh=mesh,
        out_type=jax.ShapeDtypeStruct((H + 1, B), jnp.float32),
        scratch_types=[
            pltpu.VMEM((B,), jnp.int32),
            pltpu.VMEM((1, V), jnp.float32),
            pltpu.VMEM((1, B), jnp.float32),
        ],
        compiler_params=pltpu.CompilerParams(needs_layout_passes=False),
    )
    def gather_kernel(idx_hbm, et_hbm, out_hbm, idx_v, row_v, out_v):
        wid = lax.axis_index("s") * NC + lax.axis_index("c")
        zero = jnp.full((L,), 0, jnp.int32)
        pltpu.sync_copy(idx_hbm, idx_v)
        pltpu.sync_copy(et_hbm.at[pl.ds(wid, 1)], row_v)
        for g in range(n_grp):
            ids = idx_v[pl.ds(g * L, L)]
            vals = plsc.load_gather(row_v, [zero, ids])
            plsc.store_scatter(
                out_v, [zero, lax.iota(jnp.int32, L) + g * L], vals)
        pltpu.sync_copy(out_v, out_hbm.at[pl.ds(wid, 1)])

        @pl.when(wid == 0)
        def _():
            for g in range(n_grp):
                plsc.store_scatter(
                    out_v, [zero, lax.iota(jnp.int32, L) + g * L],
                    jnp.full((L,), 1.0, jnp.float32))
            pltpu.sync_copy(out_v, out_hbm.at[pl.ds(H, 1)])

    return gather_kernel


def _proj_body(w_ref, b_ref, h_ref, out_ref):
    lhs = jnp.concatenate([w_ref[...], b_ref[...]], axis=0)
    out_ref[...] = lax.dot_general(
        lhs, h_ref[...],
        (((0,), (0,)), ((), ())),
        preferred_element_type=jnp.float32,
    )


@functools.lru_cache(maxsize=None)
def _make_tc_proj(B, H, V, v_tile):
    grid = (pl.cdiv(V, v_tile),)
    return pl.pallas_call(
        _proj_body,
        grid=grid,
        in_specs=[
            pl.BlockSpec((H, v_tile), lambda j: (0, j)),
            pl.BlockSpec((1, v_tile), lambda j: (0, j)),
            pl.BlockSpec((H + 1, B), lambda j: (0, 0)),
        ],
        out_specs=pl.BlockSpec((v_tile, B), lambda j: (j, 0)),
        out_shape=jax.ShapeDtypeStruct((V, B), jnp.float32),
        compiler_params=pltpu.CompilerParams(
            dimension_semantics=("arbitrary",),
        ),
    )


def kernel(input_ids, embed_table, proj_weight, proj_bias):
    B, = input_ids.shape
    V, H = embed_table.shape
    et_t = jnp.transpose(embed_table)
    w_t = jnp.transpose(proj_weight)
    hidden_aug_t = _make_sc_gather(V, H, B)(input_ids.astype(jnp.int32), et_t)
    logits_t = _make_tc_proj(B, H, V, 1024)(
        w_t, proj_bias.reshape(1, V), hidden_aug_t
    )
    return jnp.transpose(logits_t)


# speedup vs baseline: 3.9873x; 1.1301x over previous
import functools

import jax
import jax.numpy as jnp
from jax import lax
from jax.experimental import pallas as pl
from jax.experimental.pallas import tpu as pltpu
from jax.experimental.pallas import tpu_sc as plsc


@functools.lru_cache(maxsize=None)
def _make_sc_gather(V, H, B):
    info = plsc.get_sparse_core_info()
    NC, NS, L = info.num_cores, info.num_subcores, info.num_lanes
    NW = NC * NS
    assert H == NW and B % L == 0
    n_grp = B // L
    mesh = plsc.VectorSubcoreMesh(core_axis_name="c", subcore_axis_name="s")

    @functools.partial(
        pl.kernel,
        mesh=mesh,
        out_type=jax.ShapeDtypeStruct((H + 1, B), jnp.float32),
        scratch_types=[
            pltpu.VMEM((B,), jnp.int32),
            pltpu.VMEM((1, V), jnp.float32),
            pltpu.VMEM((1, B), jnp.float32),
        ],
        compiler_params=pltpu.CompilerParams(needs_layout_passes=False),
    )
    def gather_kernel(idx_hbm, et_hbm, out_hbm, idx_v, row_v, out_v):
        wid = lax.axis_index("s") * NC + lax.axis_index("c")
        zero = jnp.full((L,), 0, jnp.int32)
        pltpu.sync_copy(idx_hbm, idx_v)
        pltpu.sync_copy(et_hbm.at[pl.ds(wid, 1)], row_v)
        for g in range(n_grp):
            ids = idx_v[pl.ds(g * L, L)]
            vals = plsc.load_gather(row_v, [zero, ids])
            plsc.store_scatter(
                out_v, [zero, lax.iota(jnp.int32, L) + g * L], vals)
        pltpu.sync_copy(out_v, out_hbm.at[pl.ds(wid, 1)])

        @pl.when(wid == 0)
        def _():
            for g in range(n_grp):
                plsc.store_scatter(
                    out_v, [zero, lax.iota(jnp.int32, L) + g * L],
                    jnp.full((L,), 1.0, jnp.float32))
            pltpu.sync_copy(out_v, out_hbm.at[pl.ds(H, 1)])

    return gather_kernel


def _proj_body(w_ref, b_ref, h_ref, out_ref):
    lhs = jnp.concatenate([w_ref[...], b_ref[...]], axis=0)
    out_ref[...] = lax.dot_general(
        lhs, h_ref[...],
        (((0,), (0,)), ((), ())),
        preferred_element_type=jnp.float32,
    )


@functools.lru_cache(maxsize=None)
def _make_tc_proj(B, H, V, v_tile):
    grid = (pl.cdiv(V, v_tile),)
    return pl.pallas_call(
        _proj_body,
        grid=grid,
        in_specs=[
            pl.BlockSpec((H, v_tile), lambda j: (0, j)),
            pl.BlockSpec((1, v_tile), lambda j: (0, j)),
            pl.BlockSpec((H + 1, B), lambda j: (0, 0)),
        ],
        out_specs=pl.BlockSpec((v_tile, B), lambda j: (j, 0)),
        out_shape=jax.ShapeDtypeStruct((V, B), jnp.float32),
        compiler_params=pltpu.CompilerParams(
            dimension_semantics=("arbitrary",),
        ),
    )


def kernel(input_ids, embed_table, proj_weight, proj_bias):
    B, = input_ids.shape
    V, H = embed_table.shape
    et_t = jnp.transpose(embed_table)
    w_t = jnp.transpose(proj_weight)
    hidden_aug_t = _make_sc_gather(V, H, B)(input_ids.astype(jnp.int32), et_t)
    logits_t = _make_tc_proj(B, H, V, 2048)(
        w_t, proj_bias.reshape(1, V), hidden_aug_t
    )
    return jnp.transpose(logits_t)


# speedup vs baseline: 3.9951x; 1.0020x over previous
import functools

import jax
import jax.numpy as jnp
from jax import lax
from jax.experimental import pallas as pl
from jax.experimental.pallas import tpu as pltpu
from jax.experimental.pallas import tpu_sc as plsc


@functools.lru_cache(maxsize=None)
def _make_sc_gather(V, H, B):
    info = plsc.get_sparse_core_info()
    NC, NS, L = info.num_cores, info.num_subcores, info.num_lanes
    NW = NC * NS
    assert H == NW and B % L == 0
    n_grp = B // L
    mesh = plsc.VectorSubcoreMesh(core_axis_name="c", subcore_axis_name="s")

    @functools.partial(
        pl.kernel,
        mesh=mesh,
        out_type=jax.ShapeDtypeStruct((H + 1, B), jnp.float32),
        scratch_types=[
            pltpu.VMEM((B,), jnp.int32),
            pltpu.VMEM((1, V), jnp.float32),
            pltpu.VMEM((1, B), jnp.float32),
        ],
        compiler_params=pltpu.CompilerParams(needs_layout_passes=False),
    )
    def gather_kernel(idx_hbm, et_hbm, out_hbm, idx_v, row_v, out_v):
        wid = lax.axis_index("s") * NC + lax.axis_index("c")
        zero = jnp.full((L,), 0, jnp.int32)
        pltpu.sync_copy(idx_hbm, idx_v)
        pltpu.sync_copy(et_hbm.at[pl.ds(wid, 1)], row_v)
        for g in range(n_grp):
            ids = idx_v[pl.ds(g * L, L)]
            vals = plsc.load_gather(row_v, [zero, ids])
            plsc.store_scatter(
                out_v, [zero, lax.iota(jnp.int32, L) + g * L], vals)
        pltpu.sync_copy(out_v, out_hbm.at[pl.ds(wid, 1)])

        @pl.when(wid == 0)
        def _():
            for g in range(n_grp):
                plsc.store_scatter(
                    out_v, [zero, lax.iota(jnp.int32, L) + g * L],
                    jnp.full((L,), 1.0, jnp.float32))
            pltpu.sync_copy(out_v, out_hbm.at[pl.ds(H, 1)])

    return gather_kernel


def _proj_body(w_ref, b_ref, h_ref, out_ref):
    lhs = jnp.concatenate([w_ref[...], b_ref[...]], axis=0)
    out_ref[...] = lax.dot_general(
        lhs, h_ref[...],
        (((0,), (0,)), ((), ())),
        preferred_element_type=jnp.float32,
    )


@functools.lru_cache(maxsize=None)
def _make_tc_proj(B, H, V, v_tile):
    grid = (pl.cdiv(V, v_tile),)
    return pl.pallas_call(
        _proj_body,
        grid=grid,
        in_specs=[
            pl.BlockSpec((H, v_tile), lambda j: (0, j)),
            pl.BlockSpec((1, v_tile), lambda j: (0, j)),
            pl.BlockSpec((H + 1, B), lambda j: (0, 0)),
        ],
        out_specs=pl.BlockSpec((v_tile, B), lambda j: (j, 0)),
        out_shape=jax.ShapeDtypeStruct((V, B), jnp.float32),
        compiler_params=pltpu.CompilerParams(
            dimension_semantics=("arbitrary",),
        ),
    )


def kernel(input_ids, embed_table, proj_weight, proj_bias):
    B, = input_ids.shape
    V, H = embed_table.shape
    et_t = jnp.transpose(embed_table)
    w_t = jnp.transpose(proj_weight)
    hidden_aug_t = _make_sc_gather(V, H, B)(input_ids.astype(jnp.int32), et_t)
    logits_t = _make_tc_proj(B, H, V, 4096)(
        w_t, proj_bias.reshape(1, V), hidden_aug_t
    )
    return jnp.transpose(logits_t)
